# baseline (speedup 1.0000x reference)
import jax
import jax.numpy as jnp
from jax import lax
from jax.experimental import pallas as pl
from jax.experimental.pallas import tpu as pltpu
from jax.experimental.pallas import tpu_sc as plsc

_NC = 2
_NS = 16
_NW = _NC * _NS
_H = 16
_NBUF = 4


def _emb_body(ids_hbm, wte_hbm, wpe_hbm, out_hbm,
              idx_all, wpe_v, b0, b1, b2, b3,
              gs0, gs1, gs2, gs3, ws0, ws1, ws2, ws3, isem):
    B, _ = ids_hbm.shape
    P, D = wpe_v.shape
    wid = lax.axis_index("s") * _NC + lax.axis_index("c")
    p0 = wid * P
    idx_copies = [
        pltpu.make_async_copy(ids_hbm.at[b, pl.ds(p0, P)], idx_all.at[b], isem)
        for b in range(B)
    ]
    for c in idx_copies:
        c.start()
    pltpu.sync_copy(wpe_hbm.at[pl.ds(p0, P)], wpe_v)
    for c in idx_copies:
        c.wait()

    bufs = (b0, b1, b2, b3)
    gsems = (gs0, gs1, gs2, gs3)
    wsems = (ws0, ws1, ws2, ws3)

    def gather_start(k, r, m):
        idx = idx_all.at[k, pl.ds(r * _H, _H)]
        pltpu.async_copy(wte_hbm.at[idx], bufs[m], gsems[m])

    def gather_wait(m):
        pltpu.make_async_copy(wte_hbm.at[idx_all.at[0, pl.ds(0, _H)]],
                              bufs[m], gsems[m]).wait()

    def write(k, r, m):
        dst = out_hbm.at[k, pl.ds(p0 + r * _H, _H)]
        return pltpu.make_async_copy(bufs[m], dst, wsems[m])

    def accum_wpe(r, m):
        buf = bufs[m]

        def row_body(i, c):
            for j in range(D // 16):
                sl = pl.ds(j * 16, 16)
                buf[i, sl] = buf[i, sl] + wpe_v[r * _H + i, sl]
            return c

        lax.fori_loop(0, _H, row_body, 0)

    gather_start(0, 0, 0)
    gather_start(0, 1, 1)
    gather_start(1, 0, 2)
    gather_start(1, 1, 3)

    def outer_body(g, carry):
        for m in range(_NBUF):
            k = 2 * g + m // 2
            r = m % 2
            mp = (m - 1) % 4
            if m == 0:
                kp, rp, guard_prev = 2 * g - 1, 1, g > 0
            elif m == 1:
                kp, rp, guard_prev = 2 * g, 0, g < B // 2 - 1
            elif m == 2:
                kp, rp, guard_prev = 2 * g, 1, g < B // 2 - 1
            else:
                kp, rp, guard_prev = 2 * g + 1, 0, g < B // 2 - 1

            @pl.when(guard_prev)
            def _():
                write(kp, rp, mp).wait()
                gather_start(kp + 2, rp, mp)

            gather_wait(m)
            accum_wpe(r, m)
            write(k, r, m).start()
        return carry

    lax.fori_loop(0, B // 2, outer_body, 0)
    write(B - 2, 0, 0).wait()
    write(B - 2, 1, 1).wait()
    write(B - 1, 0, 2).wait()
    write(B - 1, 1, 3).wait()


def kernel(input_ids, wte, wpe):
    B, S = input_ids.shape
    V, D = wte.shape
    P = S // _NW
    mesh = plsc.VectorSubcoreMesh(
        core_axis_name="c", subcore_axis_name="s",
        num_cores=_NC, num_subcores=_NS,
    )
    f = pl.kernel(
        _emb_body,
        out_type=jax.ShapeDtypeStruct((B, S, D), jnp.float32),
        mesh=mesh,
        scratch_types=[
            pltpu.VMEM((B, P), jnp.int32),
            pltpu.VMEM((P, D), jnp.float32),
            pltpu.VMEM((_H, D), jnp.float32),
            pltpu.VMEM((_H, D), jnp.float32),
            pltpu.VMEM((_H, D), jnp.float32),
            pltpu.VMEM((_H, D), jnp.float32),
            pltpu.SemaphoreType.DMA,
            pltpu.SemaphoreType.DMA,
            pltpu.SemaphoreType.DMA,
            pltpu.SemaphoreType.DMA,
            pltpu.SemaphoreType.DMA,
            pltpu.SemaphoreType.DMA,
            pltpu.SemaphoreType.DMA,
            pltpu.SemaphoreType.DMA,
            pltpu.SemaphoreType.DMA,
        ],
    )
    return f(input_ids.astype(jnp.int32), wte, wpe)


# speedup vs baseline: 1.5673x; 1.5673x over previous
import jax
import jax.numpy as jnp
from jax import lax
from jax.experimental import pallas as pl
from jax.experimental.pallas import tpu as pltpu
from jax.experimental.pallas import tpu_sc as plsc

_NC = 2
_NS = 16
_NW = _NC * _NS
_H = 16
_NBUF = 4


def _emb_body(ids_hbm, wte_hbm, wpe_hbm, out_hbm,
              idx_all, wpe_v, b0, b1, b2, b3,
              gs0, gs1, gs2, gs3, ws0, ws1, ws2, ws3, isem):
    B, _ = ids_hbm.shape
    P, D = wpe_v.shape
    wid = lax.axis_index("s") * _NC + lax.axis_index("c")
    p0 = wid * P
    idx_copies = [
        pltpu.make_async_copy(ids_hbm.at[b, pl.ds(p0, P)], idx_all.at[b], isem)
        for b in range(B)
    ]
    for c in idx_copies:
        c.start()
    pltpu.sync_copy(wpe_hbm.at[pl.ds(p0, P)], wpe_v)
    for c in idx_copies:
        c.wait()

    bufs = (b0, b1, b2, b3)
    gsems = (gs0, gs1, gs2, gs3)
    wsems = (ws0, ws1, ws2, ws3)

    def gather_start(k, r, m):
        idx = idx_all.at[k, pl.ds(r * _H, _H)]
        pltpu.async_copy(wte_hbm.at[idx], bufs[m], gsems[m])

    def gather_wait(m):
        pltpu.make_async_copy(wte_hbm.at[idx_all.at[0, pl.ds(0, _H)]],
                              bufs[m], gsems[m]).wait()

    def write(k, r, m):
        dst = out_hbm.at[k, pl.ds(p0 + r * _H, _H)]
        return pltpu.make_async_copy(bufs[m], dst, wsems[m])

    def accum_wpe(r, m):
        buf = bufs[m]

        def row_body(i, c):
            for j in range(D // 16):
                sl = pl.ds(j * 16, 16)
                plsc.addupdate(buf.at[i, sl], wpe_v[r * _H + i, sl])
            return c

        lax.fori_loop(0, _H, row_body, 0)

    gather_start(0, 0, 0)
    gather_start(0, 1, 1)
    NG = B // 2

    def outer_body(g, carry):
        for m in range(_NBUF):
            k = 2 * g + m // 2
            r = m % 2
            gather_wait(m)
            accum_wpe(r, m)
            write(k, r, m).start()
            if m == 0:
                ww = (2 * g - 1, 0, 2, g > 0)
                gg = (2 * g + 1, 0, 2, None)
            elif m == 1:
                ww = (2 * g - 1, 1, 3, g > 0)
                gg = (2 * g + 1, 1, 3, None)
            elif m == 2:
                ww = (2 * g, 0, 0, None)
                gg = (2 * g + 2, 0, 0, g < NG - 1)
            else:
                ww = (2 * g, 1, 1, None)
                gg = (2 * g + 2, 1, 1, g < NG - 1)

            kw, rw, mw, wguard = ww
            if wguard is None:
                write(kw, rw, mw).wait()
            else:
                @pl.when(wguard)
                def _():
                    write(kw, rw, mw).wait()

            kg, rg, mg, gguard = gg
            if gguard is None:
                gather_start(kg, rg, mg)
            else:
                @pl.when(gguard)
                def _():
                    gather_start(kg, rg, mg)
        return carry

    lax.fori_loop(0, NG, outer_body, 0)
    write(B - 1, 0, 2).wait()
    write(B - 1, 1, 3).wait()


def kernel(input_ids, wte, wpe):
    B, S = input_ids.shape
    V, D = wte.shape
    P = S // _NW
    mesh = plsc.VectorSubcoreMesh(
        core_axis_name="c", subcore_axis_name="s",
        num_cores=_NC, num_subcores=_NS,
    )
    f = pl.kernel(
        _emb_body,
        out_type=jax.ShapeDtypeStruct((B, S, D), jnp.float32),
        mesh=mesh,
        scratch_types=[
            pltpu.VMEM((B, P), jnp.int32),
            pltpu.VMEM((P, D), jnp.float32),
            pltpu.VMEM((_H, D), jnp.float32),
            pltpu.VMEM((_H, D), jnp.float32),
            pltpu.VMEM((_H, D), jnp.float32),
            pltpu.VMEM((_H, D), jnp.float32),
            pltpu.SemaphoreType.DMA,
            pltpu.SemaphoreType.DMA,
            pltpu.SemaphoreType.DMA,
            pltpu.SemaphoreType.DMA,
            pltpu.SemaphoreType.DMA,
            pltpu.SemaphoreType.DMA,
            pltpu.SemaphoreType.DMA,
            pltpu.SemaphoreType.DMA,
            pltpu.SemaphoreType.DMA,
        ],
    )
    return f(input_ids.astype(jnp.int32), wte, wpe)


# speedup vs baseline: 1.9623x; 1.2520x over previous
import jax
import jax.numpy as jnp
from jax import lax
from jax.experimental import pallas as pl
from jax.experimental.pallas import tpu as pltpu
from jax.experimental.pallas import tpu_sc as plsc

_NC = 2
_NS = 16
_NW = _NC * _NS
_Q = 4
_H = 8


def _emb_body(ids_hbm, wte_hbm, wpe_hbm, out_hbm,
              idx_all, wpe_v,
              g0, g1, g2, g3, o0, o1, o2, o3,
              gs0, gs1, gs2, gs3, ws0, ws1, ws2, ws3, isem):
    B, _ = ids_hbm.shape
    P, D = wpe_v.shape
    wid = lax.axis_index("s") * _NC + lax.axis_index("c")
    p0 = wid * P
    idx_copies = [
        pltpu.make_async_copy(ids_hbm.at[b, pl.ds(p0, P)], idx_all.at[b], isem)
        for b in range(B)
    ]
    for c in idx_copies:
        c.start()
    pltpu.sync_copy(wpe_hbm.at[pl.ds(p0, P)], wpe_v)
    for c in idx_copies:
        c.wait()

    gbufs = (g0, g1, g2, g3)
    obufs = (o0, o1, o2, o3)
    gsems = (gs0, gs1, gs2, gs3)
    wsems = (ws0, ws1, ws2, ws3)

    def gather(k, q):
        idx = idx_all.at[k, pl.ds(q * _H, _H)]
        return pltpu.make_async_copy(wte_hbm.at[idx], gbufs[q], gsems[q])

    def write(k, q):
        dst = out_hbm.at[k, pl.ds(p0 + q * _H, _H)]
        return pltpu.make_async_copy(obufs[q], dst, wsems[q])

    def add_rows(q):
        g, o = gbufs[q], obufs[q]

        def row_body(i, c):
            for j in range(D // 16):
                sl = pl.ds(j * 16, 16)
                o[i, sl] = g[i, sl] + wpe_v[q * _H + i, sl]
            return c

        lax.fori_loop(0, _H, row_body, 0)

    for q in range(_Q):
        gather(0, q).start()

    def batch_body(k, carry):
        for q in range(_Q):
            gather(k, q).wait()

            @pl.when(k > 0)
            def _():
                write(k - 1, q).wait()

            add_rows(q)
            write(k, q).start()

            @pl.when(k < B - 1)
            def _():
                gather(k + 1, q).start()

        return carry

    lax.fori_loop(0, B, batch_body, 0)
    for q in range(_Q):
        write(B - 1, q).wait()


def kernel(input_ids, wte, wpe):
    B, S = input_ids.shape
    V, D = wte.shape
    P = S // _NW
    mesh = plsc.VectorSubcoreMesh(
        core_axis_name="c", subcore_axis_name="s",
        num_cores=_NC, num_subcores=_NS,
    )
    f = pl.kernel(
        _emb_body,
        out_type=jax.ShapeDtypeStruct((B, S, D), jnp.float32),
        mesh=mesh,
        scratch_types=(
            [pltpu.VMEM((B, P), jnp.int32),
             pltpu.VMEM((P, D), jnp.float32)]
            + [pltpu.VMEM((_H, D), jnp.float32) for _ in range(8)]
            + [pltpu.SemaphoreType.DMA for _ in range(9)]
        ),
    )
    return f(input_ids.astype(jnp.int32), wte, wpe)
